# Initial kernel scaffold; baseline (speedup 1.0000x reference)
import jax
import jax.numpy as jnp
from jax.experimental import pallas as pl


def kernel(x, adj, v_lin0_W, v_lin0_b, v_lins1_W, v_lins1_b, v_lins2_W, v_lins2_b, v_lins3_W, v_lins3_b, v_lins4_W, v_lins4_b, e_lin0_W, e_lin0_b, e_lins_W, e_lins_b, e_lin1_W, e_lin1_b, v_bn_g, v_bn_b, e_bn_g, e_bn_b):
    raise NotImplementedError("write your pallas kernel here")


# speedup vs baseline: 8.1383x; 8.1383x over previous
import jax
import jax.numpy as jnp
from jax import lax
from jax.experimental import pallas as pl
from jax.experimental.pallas import tpu as pltpu

_B = 64
_N = 100
_U = 64
_NN = _N * _N
_BN = _B * _N
_EREAL = _B * _N * (_N - 1)
_COORD = 100.0
_F32 = jnp.float32


def _leaky(t):
    return jnp.maximum(t, 0.01 * t)


def _diag_mask():
    r = lax.broadcasted_iota(jnp.int32, (_N, _N, _U), 0)
    c = lax.broadcasted_iota(jnp.int32, (_N, _N, _U), 1)
    return r == c


def _layer_tail(wn, x2b, x3b, x4b, Web, beb, pooled_ref, stats_ref):
    preb = jnp.dot(wn.reshape(_NN, _U), Web.T,
                   preferred_element_type=_F32).reshape(_N, _N, _U)
    preb = preb + beb + x3b[:, None, :] + x4b[None, :, :]
    dm = _diag_mask()
    pm = jnp.where(dm, 0.0, preb)
    s1 = jnp.sum(pm, axis=(0, 1))
    s2 = jnp.sum(pm * pm, axis=(0, 1))
    stats_ref[0] = jnp.stack([s1, s2])
    msg = jax.nn.sigmoid(wn) * x2b[None, :, :]
    msg = jnp.where(dm, -jnp.inf, msg)
    pooled_ref[0] = jnp.max(msg, axis=1)


def _advance(w, x3a, x4a, sc, sh, Wea, bea):
    pre = jnp.dot(w.reshape(_NN, _U), Wea.T,
                  preferred_element_type=_F32).reshape(_N, _N, _U)
    pre = pre + bea + x3a[:, None, :] + x4a[None, :, :]
    return w + _leaky(pre * sc + sh)


def _edge_first(adj_ref, e0w_ref, e0b_ref, x2_ref, x3_ref, x4_ref,
                We_ref, be_ref, w_out_ref, pooled_ref, stats_ref):
    a = adj_ref[0] * (1.0 / _COORD)
    w0 = _leaky(a * e0w_ref[0][None, None, :] + e0b_ref[0])
    w_out_ref[0] = w0
    _layer_tail(w0, x2_ref[0], x3_ref[0], x4_ref[0],
                We_ref[...], be_ref[0], pooled_ref, stats_ref)


def _edge_mid(w_ref, x3a_ref, x4a_ref, sc_ref, sh_ref, Wea_ref, bea_ref,
              x2b_ref, x3b_ref, x4b_ref, Web_ref, beb_ref,
              w_out_ref, pooled_ref, stats_ref):
    wn = _advance(w_ref[0], x3a_ref[0], x4a_ref[0], sc_ref[0],
                  sh_ref[0], Wea_ref[...], bea_ref[0])
    w_out_ref[0] = wn
    _layer_tail(wn, x2b_ref[0], x3b_ref[0], x4b_ref[0],
                Web_ref[...], beb_ref[0], pooled_ref, stats_ref)


def _edge_last(w_ref, x3a_ref, x4a_ref, sc_ref, sh_ref, Wea_ref, bea_ref,
               e1w_ref, e1b_ref, out_ref):
    wn = _advance(w_ref[0], x3a_ref[0], x4a_ref[0], sc_ref[0],
                  sh_ref[0], Wea_ref[...], bea_ref[0])
    o = jnp.sum(wn * e1w_ref[0][None, None, :], axis=-1) + e1b_ref[0, 0]
    r = lax.broadcasted_iota(jnp.int32, (_N, _N), 0)
    c = lax.broadcasted_iota(jnp.int32, (_N, _N), 1)
    out_ref[0] = jnp.where(r == c, 0.0, o)


def _proj(x0, W_ref, b_ref):
    return jnp.dot(x0, W_ref[...].T, preferred_element_type=_F32) + b_ref[0]


def _node0(x_ref, W0_ref, b0_ref, W1_ref, b1_ref, W2_ref, b2_ref,
           W3_ref, b3_ref, W4_ref, b4_ref,
           x0_ref, p1_ref, p2_ref, p3_ref, p4_ref):
    x = x_ref[...] * (1.0 / _COORD)
    x0 = _leaky(jnp.dot(x, W0_ref[...].T, preferred_element_type=_F32)
                + b0_ref[0])
    x0_ref[...] = x0
    p1_ref[...] = _proj(x0, W1_ref, b1_ref)
    p2_ref[...] = _proj(x0, W2_ref, b2_ref)
    p3_ref[...] = _proj(x0, W3_ref, b3_ref)
    p4_ref[...] = _proj(x0, W4_ref, b4_ref)


def _node_step(x_ref, p1_ref, pooled_ref, g_ref, bb_ref,
               W1_ref, b1_ref, W2_ref, b2_ref, W3_ref, b3_ref,
               W4_ref, b4_ref,
               xn_ref, q1_ref, q2_ref, q3_ref, q4_ref):
    t = p1_ref[...] + pooled_ref[...]
    m = jnp.mean(t, axis=0, keepdims=True)
    v = jnp.mean((t - m) * (t - m), axis=0, keepdims=True)
    bn = (t - m) / jnp.sqrt(v + 1e-5) * g_ref[0] + bb_ref[0]
    xn = x_ref[...] + _leaky(bn)
    xn_ref[...] = xn
    q1_ref[...] = _proj(xn, W1_ref, b1_ref)
    q2_ref[...] = _proj(xn, W2_ref, b2_ref)
    q3_ref[...] = _proj(xn, W3_ref, b3_ref)
    q4_ref[...] = _proj(xn, W4_ref, b4_ref)


_node_sds = [jax.ShapeDtypeStruct((_BN, _U), _F32)] * 5

_EB = pl.BlockSpec((1, _N, _N, _U), lambda b: (b, 0, 0, 0))
_NB = pl.BlockSpec((1, _N, _U), lambda b: (b, 0, 0))
_WB = pl.BlockSpec((_U, _U), lambda b: (0, 0))
_RB = pl.BlockSpec((1, _U), lambda b: (0, 0))
_AB = pl.BlockSpec((1, _N, _N), lambda b: (b, 0, 0))
_A4 = pl.BlockSpec((1, _N, _N, 1), lambda b: (b, 0, 0, 0))
_SB = pl.BlockSpec((1, 2, _U), lambda b: (b, 0, 0))
_CP = pltpu.CompilerParams(dimension_semantics=("parallel",))

_w_sd = jax.ShapeDtypeStruct((_B, _N, _N, _U), _F32)
_pool_sd = jax.ShapeDtypeStruct((_B, _N, _U), _F32)
_stat_sd = jax.ShapeDtypeStruct((_B, 2, _U), _F32)

_first_call = pl.pallas_call(
    _edge_first, grid=(_B,),
    in_specs=[_A4, _RB, _RB, _NB, _NB, _NB, _WB, _RB],
    out_specs=[_EB, _NB, _SB],
    out_shape=[_w_sd, _pool_sd, _stat_sd],
    compiler_params=_CP)

_mid_call = pl.pallas_call(
    _edge_mid, grid=(_B,),
    in_specs=[_EB, _NB, _NB, _RB, _RB, _WB, _RB, _NB, _NB, _NB, _WB, _RB],
    out_specs=[_EB, _NB, _SB],
    out_shape=[_w_sd, _pool_sd, _stat_sd],
    compiler_params=_CP)

_last_call = pl.pallas_call(
    _edge_last, grid=(_B,),
    in_specs=[_EB, _NB, _NB, _RB, _RB, _WB, _RB, _RB,
              pl.BlockSpec((1, 1), lambda b: (0, 0))],
    out_specs=_AB,
    out_shape=jax.ShapeDtypeStruct((_B, _N, _N), _F32),
    compiler_params=_CP)

_node0_call = pl.pallas_call(_node0, out_shape=_node_sds)
_node_step_call = pl.pallas_call(_node_step, out_shape=_node_sds)


def _bn_fold(stats, g, b):
    s = jnp.sum(stats, axis=0)
    mean = s[0] / _EREAL
    var = s[1] / _EREAL - mean * mean
    sc = g / jnp.sqrt(var + 1e-5)
    sh = b - mean * sc
    return sc[None], sh[None]


def kernel(x, adj, v_lin0_W, v_lin0_b, v_lins1_W, v_lins1_b, v_lins2_W,
           v_lins2_b, v_lins3_W, v_lins3_b, v_lins4_W, v_lins4_b,
           e_lin0_W, e_lin0_b, e_lins_W, e_lins_b, e_lin1_W, e_lin1_b,
           v_bn_g, v_bn_b, e_bn_g, e_bn_b):
    xf = jnp.pad(x.reshape(_BN, 2), ((0, 0), (0, 6)))
    W0p = jnp.pad(v_lin0_W, ((0, 0), (0, 6)))

    x0, p1, p2, p3, p4 = _node0_call(
        xf, W0p, v_lin0_b[None], v_lins1_W[0], v_lins1_b[0][None],
        v_lins2_W[0], v_lins2_b[0][None], v_lins3_W[0], v_lins3_b[0][None],
        v_lins4_W[0], v_lins4_b[0][None])

    e0w = e_lin0_W.T
    e0b = e_lin0_b[None]

    def g3(t):
        return t.reshape(_B, _N, _U)

    w, pooled, stats = _first_call(
        adj[..., None], e0w, e0b, g3(p2), g3(p3), g3(p4),
        e_lins_W[0], e_lins_b[0][None])

    for i in range(2):
        sc, sh = _bn_fold(stats, e_bn_g[i], e_bn_b[i])
        xn, q1, q2, q3, q4 = _node_step_call(
            x0, p1, pooled.reshape(_BN, _U), v_bn_g[i][None],
            v_bn_b[i][None],
            v_lins1_W[i + 1], v_lins1_b[i + 1][None],
            v_lins2_W[i + 1], v_lins2_b[i + 1][None],
            v_lins3_W[i + 1], v_lins3_b[i + 1][None],
            v_lins4_W[i + 1], v_lins4_b[i + 1][None])
        w, pooled, stats = _mid_call(
            w, g3(p3), g3(p4), sc, sh, e_lins_W[i], e_lins_b[i][None],
            g3(q2), g3(q3), g3(q4), e_lins_W[i + 1], e_lins_b[i + 1][None])
        x0, p1, p2, p3, p4 = xn, q1, q2, q3, q4

    sc, sh = _bn_fold(stats, e_bn_g[2], e_bn_b[2])
    out = _last_call(w, g3(p3), g3(p4), sc, sh, e_lins_W[2],
                     e_lins_b[2][None], e_lin1_W, e_lin1_b[None])
    return out
